# baseline (speedup 1.0000x reference)
import functools

import jax
import jax.numpy as jnp
from jax import lax
from jax.experimental import pallas as pl
from jax.experimental.pallas import tpu as pltpu
from jax.experimental.pallas import tpu_sc as plsc

_B, _L, _H = 4096, 200, 128
_NCOMBO = 12 * 7 * 2
_NC, _NS = 2, 16
_NW = _NC * _NS
_TOK = _B * _L
_ROWS_W = _TOK // _NW
_CH = 128
_NCHUNK = _ROWS_W // _CH
_LBLK = 8


def _table_body(month_ref, week_ref, hol_ref, ww_ref, bw_ref, wh_ref, bh_ref,
                pe_ref, out_ref, comb_ref):
    @pl.when(pl.program_id(0) == 0)
    def _():
        wproj = jnp.dot(week_ref[...], ww_ref[...],
                        preferred_element_type=jnp.float32) + bw_ref[...]
        hproj = jnp.dot(hol_ref[...], wh_ref[...],
                        preferred_element_type=jnp.float32) + bh_ref[...]
        r_m = lax.broadcasted_iota(jnp.int32, (_NCOMBO, 12), 0) // 14
        c_m = lax.broadcasted_iota(jnp.int32, (_NCOMBO, 12), 1)
        s_m = (r_m == c_m).astype(jnp.float32)
        r_w = (lax.broadcasted_iota(jnp.int32, (_NCOMBO, 7), 0) % 14) // 2
        c_w = lax.broadcasted_iota(jnp.int32, (_NCOMBO, 7), 1)
        s_w = (r_w == c_w).astype(jnp.float32)
        r_h = lax.broadcasted_iota(jnp.int32, (_NCOMBO, 2), 0) % 2
        c_h = lax.broadcasted_iota(jnp.int32, (_NCOMBO, 2), 1)
        s_h = (r_h == c_h).astype(jnp.float32)
        comb_ref[...] = (
            jnp.dot(s_m, month_ref[...], preferred_element_type=jnp.float32)
            + jnp.dot(s_w, wproj, preferred_element_type=jnp.float32)
            + jnp.dot(s_h, hproj, preferred_element_type=jnp.float32))

    out_ref[...] = comb_ref[...][None, :, :] + pe_ref[...][:, None, :]


def _build_table(month, week, hol, ww, bw2, wh, bh2, pe2):
    grid = _L // _LBLK
    const = lambda blk: pl.BlockSpec(blk, lambda i: tuple(0 for _ in blk))
    return pl.pallas_call(
        _table_body,
        grid=(grid,),
        in_specs=[
            const((12, _H)),
            const((7, _H // 2)),
            const((2, _H // 4)),
            const((_H // 2, _H)),
            const((1, _H)),
            const((_H // 4, _H)),
            const((1, _H)),
            pl.BlockSpec((_LBLK, _H), lambda i: (i, 0)),
        ],
        out_specs=pl.BlockSpec((_LBLK, _NCOMBO, _H), lambda i: (i, 0, 0)),
        out_shape=jax.ShapeDtypeStruct((_L, _NCOMBO, _H), jnp.float32),
        scratch_shapes=[pltpu.VMEM((_NCOMBO, _H), jnp.float32)],
    )(month, week, hol, ww, bw2, wh, bh2, pe2)


def _idx_body(m_ref, w_ref, h_ref, out_ref):
    rows, cols = m_ref.shape
    row = lax.broadcasted_iota(jnp.int32, (rows, cols), 0)
    col = lax.broadcasted_iota(jnp.int32, (rows, cols), 1)
    flat = (pl.program_id(0) * rows + row) * cols + col
    l = flat % _L
    out_ref[...] = (l * _NCOMBO + (m_ref[...] - 1) * 14
                    + w_ref[...] * 2 + h_ref[...])


def _build_idx(m2, w2, h2):
    rows = m2.shape[0]
    rblk = 400
    grid = rows // rblk
    spec = pl.BlockSpec((rblk, 128), lambda i: (i, 0))
    return pl.pallas_call(
        _idx_body,
        grid=(grid,),
        in_specs=[spec, spec, spec],
        out_specs=spec,
        out_shape=jax.ShapeDtypeStruct((rows, 128), jnp.int32),
    )(m2, w2, h2)


def _gather_body(nchunk, t_hbm, g_hbm, out_hbm, idx_v, rows_v,
                 sg0, sg1, sg2, sg3, so0, so1, so2, so3):
    wid = lax.axis_index("s") * _NC + lax.axis_index("c")
    pltpu.sync_copy(g_hbm.at[pl.ds(wid * nchunk, nchunk)], idx_v)
    base = wid * nchunk * _CH
    sg = (sg0, sg1, sg2, sg3)
    so = (so0, so1, so2, so3)

    def start_gather(i, b):
        pltpu.async_copy(t_hbm.at[idx_v.at[i]], rows_v.at[b], sg[b])

    def wait_gather(b):
        pltpu.make_async_copy(t_hbm.at[pl.ds(0, _CH)], rows_v.at[b],
                              sg[b]).wait()

    def start_out(i, b):
        pltpu.async_copy(rows_v.at[b], out_hbm.at[pl.ds(base + i * _CH, _CH)],
                         so[b])

    def wait_out(b):
        pltpu.make_async_copy(rows_v.at[b], out_hbm.at[pl.ds(base, _CH)],
                              so[b]).wait()

    start_gather(0, 0)
    start_gather(1, 1)
    start_gather(2, 2)
    wait_gather(0)
    start_out(0, 0)
    start_gather(3, 3)
    wait_gather(1)
    start_out(1, 1)

    def body2(g, carry):
        for b4 in range(4):
            i = 2 + g * 4 + b4
            bcur = (2 + b4) % 4
            bnext = b4
            wait_out(bnext)
            start_gather(i + 2, bnext)
            wait_gather(bcur)
            start_out(i, bcur)
        return carry

    lax.fori_loop(0, (nchunk - 4) // 4, body2, 0)

    wait_gather((nchunk - 2) % 4)
    start_out(nchunk - 2, (nchunk - 2) % 4)
    wait_gather((nchunk - 1) % 4)
    start_out(nchunk - 1, (nchunk - 1) % 4)
    for b in range(4):
        wait_out(b)


def _sc_gather(t2, gidx2, n_tok):
    nchunk = n_tok // (_NW * _CH)
    mesh = plsc.VectorSubcoreMesh(core_axis_name="c", subcore_axis_name="s")
    fn = pl.kernel(
        functools.partial(_gather_body, nchunk),
        mesh=mesh,
        out_type=jax.ShapeDtypeStruct((n_tok, _H), jnp.float32),
        scratch_types=[
            pltpu.VMEM((nchunk, _CH), jnp.int32),
            pltpu.VMEM((4, _CH, _H), jnp.float32),
        ] + [pltpu.SemaphoreType.DMA] * 8,
    )
    return fn(t2, gidx2)


_K = 256
_TN = 1024


def _tc_body(m_ref, w_ref, h_ref, month_ref, week_ref, hol_ref, ww_ref,
             bw_ref, wh_ref, bh_ref, pe_ref, out_ref, stack_ref):
    @pl.when(pl.program_id(0) == 0)
    def _():
        stack_ref[...] = jnp.zeros((_K, _H), jnp.float32)
        stack_ref[0:12, :] = month_ref[...]
        stack_ref[16:23, :] = jnp.dot(week_ref[...], ww_ref[...],
                                      preferred_element_type=jnp.float32)
        stack_ref[24:26, :] = jnp.dot(hol_ref[...], wh_ref[...],
                                      preferred_element_type=jnp.float32)
        stack_ref[32:232, :] = pe_ref[...] + bw_ref[...] + bh_ref[...]

    t0 = _SC_TOK + pl.program_id(0) * _TN
    row = lax.broadcasted_iota(jnp.int32, (_TN, _K), 0)
    col = lax.broadcasted_iota(jnp.int32, (_TN, _K), 1)
    l = (t0 + row) % _L
    oh = ((col == m_ref[...] - 1) | (col == w_ref[...] + 16)
          | (col == h_ref[...] + 24) | (col == l + 32)).astype(jnp.float32)
    out_ref[...] = jnp.dot(oh, stack_ref[...],
                           preferred_element_type=jnp.float32)


def _tc_lookup(m1, w1, h1, month, week, hol, ww, bw2, wh, bh2, pe2):
    n = m1.shape[0]
    grid = n // _TN
    tok = pl.BlockSpec((_TN, 1), lambda i: (i, 0))
    const = lambda blk: pl.BlockSpec(blk, lambda i: tuple(0 for _ in blk))
    return pl.pallas_call(
        _tc_body,
        grid=(grid,),
        in_specs=[
            tok, tok, tok,
            const((12, _H)),
            const((7, _H // 2)),
            const((2, _H // 4)),
            const((_H // 2, _H)),
            const((1, _H)),
            const((_H // 4, _H)),
            const((1, _H)),
            const((_L, _H)),
        ],
        out_specs=pl.BlockSpec((_TN, _H), lambda i: (i, 0)),
        out_shape=jax.ShapeDtypeStruct((n, _H), jnp.float32),
        scratch_shapes=[pltpu.VMEM((_K, _H), jnp.float32)],
    )(m1, w1, h1, month, week, hol, ww, bw2, wh, bh2, pe2)


_SC_TOK = 0


def kernel(time_features, month_table, week_table, holiday_table,
           W_week, b_week, W_holiday, b_holiday, pe):
    tf = time_features.astype(jnp.int32)
    pe2 = pe[0, :_L, :]
    bw2 = b_week.reshape(1, _H)
    bh2 = b_holiday.reshape(1, _H)
    pieces = []
    if _SC_TOK:
        m2 = tf[..., 0].reshape(_TOK // 128, 128)[: _SC_TOK // 128]
        w2 = tf[..., 1].reshape(_TOK // 128, 128)[: _SC_TOK // 128]
        h2 = tf[..., 2].reshape(_TOK // 128, 128)[: _SC_TOK // 128]
        t3 = _build_table(month_table, week_table, holiday_table,
                          W_week, bw2, W_holiday, bh2, pe2)
        t2 = t3.reshape(_L * _NCOMBO, _H)
        gidx2 = _build_idx(m2, w2, h2)
        pieces.append(_sc_gather(t2, gidx2, _SC_TOK))
    if _SC_TOK < _TOK:
        m1 = tf[..., 0].reshape(_TOK, 1)[_SC_TOK:]
        w1 = tf[..., 1].reshape(_TOK, 1)[_SC_TOK:]
        h1 = tf[..., 2].reshape(_TOK, 1)[_SC_TOK:]
        pieces.append(_tc_lookup(m1, w1, h1, month_table, week_table,
                                 holiday_table, W_week, bw2, W_holiday, bh2,
                                 pe2))
    out2 = pieces[0] if len(pieces) == 1 else jnp.concatenate(pieces, axis=0)
    return out2.reshape(_B, _L, _H)


# speedup vs baseline: 1.1422x; 1.1422x over previous
import functools

import jax
import jax.numpy as jnp
from jax import lax
from jax.experimental import pallas as pl
from jax.experimental.pallas import tpu as pltpu
from jax.experimental.pallas import tpu_sc as plsc

_B, _L, _H = 4096, 200, 128
_NCOMBO = 12 * 7 * 2
_NC, _NS = 2, 16
_NW = _NC * _NS
_TOK = _B * _L
_ROWS_W = _TOK // _NW
_CH = 128
_NCHUNK = _ROWS_W // _CH
_LBLK = 8


def _table_body(month_ref, week_ref, hol_ref, ww_ref, bw_ref, wh_ref, bh_ref,
                pe_ref, out_ref, comb_ref):
    @pl.when(pl.program_id(0) == 0)
    def _():
        wproj = jnp.dot(week_ref[...], ww_ref[...],
                        preferred_element_type=jnp.float32) + bw_ref[...]
        hproj = jnp.dot(hol_ref[...], wh_ref[...],
                        preferred_element_type=jnp.float32) + bh_ref[...]
        r_m = lax.broadcasted_iota(jnp.int32, (_NCOMBO, 12), 0) // 14
        c_m = lax.broadcasted_iota(jnp.int32, (_NCOMBO, 12), 1)
        s_m = (r_m == c_m).astype(jnp.float32)
        r_w = (lax.broadcasted_iota(jnp.int32, (_NCOMBO, 7), 0) % 14) // 2
        c_w = lax.broadcasted_iota(jnp.int32, (_NCOMBO, 7), 1)
        s_w = (r_w == c_w).astype(jnp.float32)
        r_h = lax.broadcasted_iota(jnp.int32, (_NCOMBO, 2), 0) % 2
        c_h = lax.broadcasted_iota(jnp.int32, (_NCOMBO, 2), 1)
        s_h = (r_h == c_h).astype(jnp.float32)
        comb_ref[...] = (
            jnp.dot(s_m, month_ref[...], preferred_element_type=jnp.float32)
            + jnp.dot(s_w, wproj, preferred_element_type=jnp.float32)
            + jnp.dot(s_h, hproj, preferred_element_type=jnp.float32))

    out_ref[...] = comb_ref[...][None, :, :] + pe_ref[...][:, None, :]


def _build_table(month, week, hol, ww, bw2, wh, bh2, pe2):
    grid = _L // _LBLK
    const = lambda blk: pl.BlockSpec(blk, lambda i: tuple(0 for _ in blk))
    return pl.pallas_call(
        _table_body,
        grid=(grid,),
        in_specs=[
            const((12, _H)),
            const((7, _H // 2)),
            const((2, _H // 4)),
            const((_H // 2, _H)),
            const((1, _H)),
            const((_H // 4, _H)),
            const((1, _H)),
            pl.BlockSpec((_LBLK, _H), lambda i: (i, 0)),
        ],
        out_specs=pl.BlockSpec((_LBLK, _NCOMBO, _H), lambda i: (i, 0, 0)),
        out_shape=jax.ShapeDtypeStruct((_L, _NCOMBO, _H), jnp.float32),
        scratch_shapes=[pltpu.VMEM((_NCOMBO, _H), jnp.float32)],
    )(month, week, hol, ww, bw2, wh, bh2, pe2)


def _idx_body(m_ref, w_ref, h_ref, out_ref):
    rows, cols = m_ref.shape
    row = lax.broadcasted_iota(jnp.int32, (rows, cols), 0)
    col = lax.broadcasted_iota(jnp.int32, (rows, cols), 1)
    flat = (pl.program_id(0) * rows + row) * cols + col
    l = flat % _L
    out_ref[...] = (l * _NCOMBO + (m_ref[...] - 1) * 14
                    + w_ref[...] * 2 + h_ref[...])


def _build_idx(m2, w2, h2):
    rows = m2.shape[0]
    rblk = 400
    grid = rows // rblk
    spec = pl.BlockSpec((rblk, 128), lambda i: (i, 0))
    return pl.pallas_call(
        _idx_body,
        grid=(grid,),
        in_specs=[spec, spec, spec],
        out_specs=spec,
        out_shape=jax.ShapeDtypeStruct((rows, 128), jnp.int32),
    )(m2, w2, h2)


def _gather_body(nchunk, t_hbm, g_hbm, out_hbm, idx_v, rows_v,
                 sg0, sg1, sg2, sg3, so0, so1, so2, so3):
    wid = lax.axis_index("s") * _NC + lax.axis_index("c")
    pltpu.sync_copy(g_hbm.at[pl.ds(wid * nchunk, nchunk)], idx_v)
    base = wid * nchunk * _CH
    sg = (sg0, sg1, sg2, sg3)
    so = (so0, so1, so2, so3)

    def start_gather(i, b):
        pltpu.async_copy(t_hbm.at[idx_v.at[i]], rows_v.at[b], sg[b])

    def wait_gather(b):
        pltpu.make_async_copy(t_hbm.at[pl.ds(0, _CH)], rows_v.at[b],
                              sg[b]).wait()

    def start_out(i, b):
        pltpu.async_copy(rows_v.at[b], out_hbm.at[pl.ds(base + i * _CH, _CH)],
                         so[b])

    def wait_out(b):
        pltpu.make_async_copy(rows_v.at[b], out_hbm.at[pl.ds(base, _CH)],
                              so[b]).wait()

    start_gather(0, 0)
    start_gather(1, 1)
    start_gather(2, 2)
    wait_gather(0)
    start_out(0, 0)
    start_gather(3, 3)
    wait_gather(1)
    start_out(1, 1)

    def body2(g, carry):
        for b4 in range(4):
            i = 2 + g * 4 + b4
            bcur = (2 + b4) % 4
            bnext = b4
            wait_out(bnext)
            start_gather(i + 2, bnext)
            wait_gather(bcur)
            start_out(i, bcur)
        return carry

    lax.fori_loop(0, (nchunk - 4) // 4, body2, 0)

    wait_gather((nchunk - 2) % 4)
    start_out(nchunk - 2, (nchunk - 2) % 4)
    wait_gather((nchunk - 1) % 4)
    start_out(nchunk - 1, (nchunk - 1) % 4)
    for b in range(4):
        wait_out(b)


def _sc_gather(t2, gidx2, n_tok):
    nchunk = n_tok // (_NW * _CH)
    mesh = plsc.VectorSubcoreMesh(core_axis_name="c", subcore_axis_name="s")
    fn = pl.kernel(
        functools.partial(_gather_body, nchunk),
        mesh=mesh,
        out_type=jax.ShapeDtypeStruct((n_tok, _H), jnp.float32),
        scratch_types=[
            pltpu.VMEM((nchunk, _CH), jnp.int32),
            pltpu.VMEM((4, _CH, _H), jnp.float32),
        ] + [pltpu.SemaphoreType.DMA] * 8,
    )
    return fn(t2, gidx2)


_K = 256
_TN = 2048


def _tc_body(m_ref, w_ref, h_ref, month_ref, week_ref, hol_ref, ww_ref,
             bw_ref, wh_ref, bh_ref, pe_ref, out_ref, stack_ref):
    @pl.when(pl.program_id(0) == 0)
    def _():
        stack_ref[...] = jnp.zeros((_K, _H), jnp.float32)
        stack_ref[0:12, :] = month_ref[...]
        stack_ref[16:23, :] = jnp.dot(week_ref[...], ww_ref[...],
                                      preferred_element_type=jnp.float32)
        stack_ref[24:26, :] = jnp.dot(hol_ref[...], wh_ref[...],
                                      preferred_element_type=jnp.float32)
        stack_ref[32:232, :] = pe_ref[...] + bw_ref[...] + bh_ref[...]

    t0 = _SC_TOK + pl.program_id(0) * _TN
    row = lax.broadcasted_iota(jnp.int32, (_TN, _K), 0)
    col = lax.broadcasted_iota(jnp.int32, (_TN, _K), 1)
    l = (t0 + row) % _L
    oh = ((col == m_ref[...] - 1) | (col == w_ref[...] + 16)
          | (col == h_ref[...] + 24) | (col == l + 32)).astype(jnp.bfloat16)
    out_ref[...] = jnp.dot(oh, stack_ref[...].astype(jnp.bfloat16),
                           preferred_element_type=jnp.float32)


def _tc_lookup(m1, w1, h1, month, week, hol, ww, bw2, wh, bh2, pe2):
    n = m1.shape[0]
    grid = n // _TN
    tok = pl.BlockSpec((_TN, 1), lambda i: (i, 0))
    const = lambda blk: pl.BlockSpec(blk, lambda i: tuple(0 for _ in blk))
    return pl.pallas_call(
        _tc_body,
        grid=(grid,),
        in_specs=[
            tok, tok, tok,
            const((12, _H)),
            const((7, _H // 2)),
            const((2, _H // 4)),
            const((_H // 2, _H)),
            const((1, _H)),
            const((_H // 4, _H)),
            const((1, _H)),
            const((_L, _H)),
        ],
        out_specs=pl.BlockSpec((_TN, _H), lambda i: (i, 0)),
        out_shape=jax.ShapeDtypeStruct((n, _H), jnp.float32),
        scratch_shapes=[pltpu.VMEM((_K, _H), jnp.float32)],
    )(m1, w1, h1, month, week, hol, ww, bw2, wh, bh2, pe2)


_SC_TOK = 0


def kernel(time_features, month_table, week_table, holiday_table,
           W_week, b_week, W_holiday, b_holiday, pe):
    tf = time_features.astype(jnp.int32)
    pe2 = pe[0, :_L, :]
    bw2 = b_week.reshape(1, _H)
    bh2 = b_holiday.reshape(1, _H)
    pieces = []
    if _SC_TOK:
        m2 = tf[..., 0].reshape(_TOK // 128, 128)[: _SC_TOK // 128]
        w2 = tf[..., 1].reshape(_TOK // 128, 128)[: _SC_TOK // 128]
        h2 = tf[..., 2].reshape(_TOK // 128, 128)[: _SC_TOK // 128]
        t3 = _build_table(month_table, week_table, holiday_table,
                          W_week, bw2, W_holiday, bh2, pe2)
        t2 = t3.reshape(_L * _NCOMBO, _H)
        gidx2 = _build_idx(m2, w2, h2)
        pieces.append(_sc_gather(t2, gidx2, _SC_TOK))
    if _SC_TOK < _TOK:
        m1 = tf[..., 0].reshape(_TOK, 1)[_SC_TOK:]
        w1 = tf[..., 1].reshape(_TOK, 1)[_SC_TOK:]
        h1 = tf[..., 2].reshape(_TOK, 1)[_SC_TOK:]
        pieces.append(_tc_lookup(m1, w1, h1, month_table, week_table,
                                 holiday_table, W_week, bw2, W_holiday, bh2,
                                 pe2))
    out2 = pieces[0] if len(pieces) == 1 else jnp.concatenate(pieces, axis=0)
    return out2.reshape(_B, _L, _H)


# speedup vs baseline: 1.4854x; 1.3005x over previous
import functools

import jax
import jax.numpy as jnp
from jax import lax
from jax.experimental import pallas as pl
from jax.experimental.pallas import tpu as pltpu
from jax.experimental.pallas import tpu_sc as plsc

_B, _L, _H = 4096, 200, 128
_NCOMBO = 12 * 7 * 2
_NC, _NS = 2, 16
_NW = _NC * _NS
_TOK = _B * _L
_ROWS_W = _TOK // _NW
_CH = 128
_NCHUNK = _ROWS_W // _CH
_LBLK = 8


def _table_body(month_ref, week_ref, hol_ref, ww_ref, bw_ref, wh_ref, bh_ref,
                pe_ref, out_ref, comb_ref):
    @pl.when(pl.program_id(0) == 0)
    def _():
        wproj = jnp.dot(week_ref[...], ww_ref[...],
                        preferred_element_type=jnp.float32) + bw_ref[...]
        hproj = jnp.dot(hol_ref[...], wh_ref[...],
                        preferred_element_type=jnp.float32) + bh_ref[...]
        r_m = lax.broadcasted_iota(jnp.int32, (_NCOMBO, 12), 0) // 14
        c_m = lax.broadcasted_iota(jnp.int32, (_NCOMBO, 12), 1)
        s_m = (r_m == c_m).astype(jnp.float32)
        r_w = (lax.broadcasted_iota(jnp.int32, (_NCOMBO, 7), 0) % 14) // 2
        c_w = lax.broadcasted_iota(jnp.int32, (_NCOMBO, 7), 1)
        s_w = (r_w == c_w).astype(jnp.float32)
        r_h = lax.broadcasted_iota(jnp.int32, (_NCOMBO, 2), 0) % 2
        c_h = lax.broadcasted_iota(jnp.int32, (_NCOMBO, 2), 1)
        s_h = (r_h == c_h).astype(jnp.float32)
        comb_ref[...] = (
            jnp.dot(s_m, month_ref[...], preferred_element_type=jnp.float32)
            + jnp.dot(s_w, wproj, preferred_element_type=jnp.float32)
            + jnp.dot(s_h, hproj, preferred_element_type=jnp.float32))

    out_ref[...] = comb_ref[...][None, :, :] + pe_ref[...][:, None, :]


def _build_table(month, week, hol, ww, bw2, wh, bh2, pe2):
    grid = _L // _LBLK
    const = lambda blk: pl.BlockSpec(blk, lambda i: tuple(0 for _ in blk))
    return pl.pallas_call(
        _table_body,
        grid=(grid,),
        in_specs=[
            const((12, _H)),
            const((7, _H // 2)),
            const((2, _H // 4)),
            const((_H // 2, _H)),
            const((1, _H)),
            const((_H // 4, _H)),
            const((1, _H)),
            pl.BlockSpec((_LBLK, _H), lambda i: (i, 0)),
        ],
        out_specs=pl.BlockSpec((_LBLK, _NCOMBO, _H), lambda i: (i, 0, 0)),
        out_shape=jax.ShapeDtypeStruct((_L, _NCOMBO, _H), jnp.float32),
        scratch_shapes=[pltpu.VMEM((_NCOMBO, _H), jnp.float32)],
    )(month, week, hol, ww, bw2, wh, bh2, pe2)


def _idx_body(m_ref, w_ref, h_ref, out_ref):
    rows, cols = m_ref.shape
    row = lax.broadcasted_iota(jnp.int32, (rows, cols), 0)
    col = lax.broadcasted_iota(jnp.int32, (rows, cols), 1)
    flat = (pl.program_id(0) * rows + row) * cols + col
    l = flat % _L
    out_ref[...] = (l * _NCOMBO + (m_ref[...] - 1) * 14
                    + w_ref[...] * 2 + h_ref[...])


def _build_idx(m2, w2, h2):
    rows = m2.shape[0]
    rblk = 400
    grid = rows // rblk
    spec = pl.BlockSpec((rblk, 128), lambda i: (i, 0))
    return pl.pallas_call(
        _idx_body,
        grid=(grid,),
        in_specs=[spec, spec, spec],
        out_specs=spec,
        out_shape=jax.ShapeDtypeStruct((rows, 128), jnp.int32),
    )(m2, w2, h2)


def _gather_body(nchunk, t_hbm, g_hbm, out_hbm, idx_v, rows_v,
                 sg0, sg1, sg2, sg3, so0, so1, so2, so3):
    wid = lax.axis_index("s") * _NC + lax.axis_index("c")
    pltpu.sync_copy(g_hbm.at[pl.ds(wid * nchunk, nchunk)], idx_v)
    base = wid * nchunk * _CH
    sg = (sg0, sg1, sg2, sg3)
    so = (so0, so1, so2, so3)

    def start_gather(i, b):
        pltpu.async_copy(t_hbm.at[idx_v.at[i]], rows_v.at[b], sg[b])

    def wait_gather(b):
        pltpu.make_async_copy(t_hbm.at[pl.ds(0, _CH)], rows_v.at[b],
                              sg[b]).wait()

    def start_out(i, b):
        pltpu.async_copy(rows_v.at[b], out_hbm.at[pl.ds(base + i * _CH, _CH)],
                         so[b])

    def wait_out(b):
        pltpu.make_async_copy(rows_v.at[b], out_hbm.at[pl.ds(base, _CH)],
                              so[b]).wait()

    start_gather(0, 0)
    start_gather(1, 1)
    start_gather(2, 2)
    wait_gather(0)
    start_out(0, 0)
    start_gather(3, 3)
    wait_gather(1)
    start_out(1, 1)

    def body2(g, carry):
        for b4 in range(4):
            i = 2 + g * 4 + b4
            bcur = (2 + b4) % 4
            bnext = b4
            wait_out(bnext)
            start_gather(i + 2, bnext)
            wait_gather(bcur)
            start_out(i, bcur)
        return carry

    lax.fori_loop(0, (nchunk - 4) // 4, body2, 0)

    wait_gather((nchunk - 2) % 4)
    start_out(nchunk - 2, (nchunk - 2) % 4)
    wait_gather((nchunk - 1) % 4)
    start_out(nchunk - 1, (nchunk - 1) % 4)
    for b in range(4):
        wait_out(b)


def _sc_gather(t2, gidx2, n_tok):
    nchunk = n_tok // (_NW * _CH)
    mesh = plsc.VectorSubcoreMesh(core_axis_name="c", subcore_axis_name="s")
    fn = pl.kernel(
        functools.partial(_gather_body, nchunk),
        mesh=mesh,
        out_type=jax.ShapeDtypeStruct((n_tok, _H), jnp.float32),
        scratch_types=[
            pltpu.VMEM((nchunk, _CH), jnp.int32),
            pltpu.VMEM((4, _CH, _H), jnp.float32),
        ] + [pltpu.SemaphoreType.DMA] * 8,
    )
    return fn(t2, gidx2)


_K = 256
_TN = 3200


def _tc_body(m_ref, w_ref, h_ref, month_ref, week_ref, hol_ref, ww_ref,
             bw_ref, wh_ref, bh_ref, pe_ref, out_ref, stack_ref, pes_ref):
    @pl.when(pl.program_id(0) == 0)
    def _():
        wproj = jnp.dot(week_ref[...], ww_ref[...],
                        preferred_element_type=jnp.float32)
        hproj = jnp.dot(hol_ref[...], wh_ref[...],
                        preferred_element_type=jnp.float32)
        r_m = lax.broadcasted_iota(jnp.int32, (_K, 12), 0) // 14
        c_m = lax.broadcasted_iota(jnp.int32, (_K, 12), 1)
        r_w = (lax.broadcasted_iota(jnp.int32, (_K, 7), 0) % 14) // 2
        c_w = lax.broadcasted_iota(jnp.int32, (_K, 7), 1)
        r_h = lax.broadcasted_iota(jnp.int32, (_K, 2), 0) % 2
        c_h = lax.broadcasted_iota(jnp.int32, (_K, 2), 1)
        valid = lax.broadcasted_iota(jnp.int32, (_K, 1), 0) < _NCOMBO
        stack_ref[...] = jnp.where(
            valid,
            jnp.dot((r_m == c_m).astype(jnp.float32), month_ref[...],
                    preferred_element_type=jnp.float32)
            + jnp.dot((r_w == c_w).astype(jnp.float32), wproj,
                      preferred_element_type=jnp.float32)
            + jnp.dot((r_h == c_h).astype(jnp.float32), hproj,
                      preferred_element_type=jnp.float32),
            0.0)
        pebb = pe_ref[...] + bw_ref[...] + bh_ref[...]
        for j in range(_TN // _L):
            pes_ref[j * _L:(j + 1) * _L, :] = pebb

    col = lax.broadcasted_iota(jnp.int32, (_TN, _K), 1)
    c = (m_ref[...] - 1) * 14 + w_ref[...] * 2 + h_ref[...]
    oh = (col == c).astype(jnp.bfloat16)
    out_ref[...] = jnp.dot(oh, stack_ref[...].astype(jnp.bfloat16),
                           preferred_element_type=jnp.float32) + pes_ref[...]


def _tc_lookup(m1, w1, h1, month, week, hol, ww, bw2, wh, bh2, pe2):
    n = m1.shape[0]
    grid = n // _TN
    tok = pl.BlockSpec((_TN, 1), lambda i: (i, 0))
    const = lambda blk: pl.BlockSpec(blk, lambda i: tuple(0 for _ in blk))
    return pl.pallas_call(
        _tc_body,
        grid=(grid,),
        in_specs=[
            tok, tok, tok,
            const((12, _H)),
            const((7, _H // 2)),
            const((2, _H // 4)),
            const((_H // 2, _H)),
            const((1, _H)),
            const((_H // 4, _H)),
            const((1, _H)),
            const((_L, _H)),
        ],
        out_specs=pl.BlockSpec((_TN, _H), lambda i: (i, 0)),
        out_shape=jax.ShapeDtypeStruct((n, _H), jnp.float32),
        scratch_shapes=[pltpu.VMEM((_K, _H), jnp.float32),
                        pltpu.VMEM((_TN, _H), jnp.float32)],
    )(m1, w1, h1, month, week, hol, ww, bw2, wh, bh2, pe2)


_SC_TOK = 0


def kernel(time_features, month_table, week_table, holiday_table,
           W_week, b_week, W_holiday, b_holiday, pe):
    tf = time_features.astype(jnp.int32)
    pe2 = pe[0, :_L, :]
    bw2 = b_week.reshape(1, _H)
    bh2 = b_holiday.reshape(1, _H)
    pieces = []
    if _SC_TOK:
        m2 = tf[..., 0].reshape(_TOK // 128, 128)[: _SC_TOK // 128]
        w2 = tf[..., 1].reshape(_TOK // 128, 128)[: _SC_TOK // 128]
        h2 = tf[..., 2].reshape(_TOK // 128, 128)[: _SC_TOK // 128]
        t3 = _build_table(month_table, week_table, holiday_table,
                          W_week, bw2, W_holiday, bh2, pe2)
        t2 = t3.reshape(_L * _NCOMBO, _H)
        gidx2 = _build_idx(m2, w2, h2)
        pieces.append(_sc_gather(t2, gidx2, _SC_TOK))
    if _SC_TOK < _TOK:
        m1 = tf[..., 0].reshape(_TOK, 1)[_SC_TOK:]
        w1 = tf[..., 1].reshape(_TOK, 1)[_SC_TOK:]
        h1 = tf[..., 2].reshape(_TOK, 1)[_SC_TOK:]
        pieces.append(_tc_lookup(m1, w1, h1, month_table, week_table,
                                 holiday_table, W_week, bw2, W_holiday, bh2,
                                 pe2))
    out2 = pieces[0] if len(pieces) == 1 else jnp.concatenate(pieces, axis=0)
    return out2.reshape(_B, _L, _H)


# speedup vs baseline: 5.2324x; 3.5226x over previous
import functools

import jax
import jax.numpy as jnp
from jax import lax
from jax.experimental import pallas as pl
from jax.experimental.pallas import tpu as pltpu
from jax.experimental.pallas import tpu_sc as plsc

_B, _L, _H = 4096, 200, 128
_NCOMBO = 12 * 7 * 2
_NC, _NS = 2, 16
_NW = _NC * _NS
_TOK = _B * _L
_ROWS_W = _TOK // _NW
_CH = 128
_NCHUNK = _ROWS_W // _CH
_LBLK = 8


def _table_body(month_ref, week_ref, hol_ref, ww_ref, bw_ref, wh_ref, bh_ref,
                pe_ref, out_ref, comb_ref):
    @pl.when(pl.program_id(0) == 0)
    def _():
        wproj = jnp.dot(week_ref[...], ww_ref[...],
                        preferred_element_type=jnp.float32) + bw_ref[...]
        hproj = jnp.dot(hol_ref[...], wh_ref[...],
                        preferred_element_type=jnp.float32) + bh_ref[...]
        r_m = lax.broadcasted_iota(jnp.int32, (_NCOMBO, 12), 0) // 14
        c_m = lax.broadcasted_iota(jnp.int32, (_NCOMBO, 12), 1)
        s_m = (r_m == c_m).astype(jnp.float32)
        r_w = (lax.broadcasted_iota(jnp.int32, (_NCOMBO, 7), 0) % 14) // 2
        c_w = lax.broadcasted_iota(jnp.int32, (_NCOMBO, 7), 1)
        s_w = (r_w == c_w).astype(jnp.float32)
        r_h = lax.broadcasted_iota(jnp.int32, (_NCOMBO, 2), 0) % 2
        c_h = lax.broadcasted_iota(jnp.int32, (_NCOMBO, 2), 1)
        s_h = (r_h == c_h).astype(jnp.float32)
        comb_ref[...] = (
            jnp.dot(s_m, month_ref[...], preferred_element_type=jnp.float32)
            + jnp.dot(s_w, wproj, preferred_element_type=jnp.float32)
            + jnp.dot(s_h, hproj, preferred_element_type=jnp.float32))

    out_ref[...] = comb_ref[...][None, :, :] + pe_ref[...][:, None, :]


def _build_table(month, week, hol, ww, bw2, wh, bh2, pe2):
    grid = _L // _LBLK
    const = lambda blk: pl.BlockSpec(blk, lambda i: tuple(0 for _ in blk))
    return pl.pallas_call(
        _table_body,
        grid=(grid,),
        in_specs=[
            const((12, _H)),
            const((7, _H // 2)),
            const((2, _H // 4)),
            const((_H // 2, _H)),
            const((1, _H)),
            const((_H // 4, _H)),
            const((1, _H)),
            pl.BlockSpec((_LBLK, _H), lambda i: (i, 0)),
        ],
        out_specs=pl.BlockSpec((_LBLK, _NCOMBO, _H), lambda i: (i, 0, 0)),
        out_shape=jax.ShapeDtypeStruct((_L, _NCOMBO, _H), jnp.float32),
        scratch_shapes=[pltpu.VMEM((_NCOMBO, _H), jnp.float32)],
    )(month, week, hol, ww, bw2, wh, bh2, pe2)


def _idx_body(with_l, m_ref, w_ref, h_ref, out_ref):
    rows, cols = m_ref.shape
    c = (m_ref[...] - 1) * 14 + w_ref[...] * 2 + h_ref[...]
    if with_l:
        row = lax.broadcasted_iota(jnp.int32, (rows, cols), 0)
        col = lax.broadcasted_iota(jnp.int32, (rows, cols), 1)
        flat = (pl.program_id(0) * rows + row) * cols + col
        c = (flat % _L) * _NCOMBO + c
    out_ref[...] = c


def _build_idx(m2, w2, h2, with_l):
    rows = m2.shape[0]
    rblk = 400
    grid = rows // rblk
    spec = pl.BlockSpec((rblk, 128), lambda i: (i, 0))
    return pl.pallas_call(
        functools.partial(_idx_body, with_l),
        grid=(grid,),
        in_specs=[spec, spec, spec],
        out_specs=spec,
        out_shape=jax.ShapeDtypeStruct((rows, 128), jnp.int32),
    )(m2, w2, h2)


def _gather_body(nchunk, t_hbm, g_hbm, out_hbm, idx_v, rows_v,
                 sg0, sg1, sg2, sg3, so0, so1, so2, so3):
    wid = lax.axis_index("s") * _NC + lax.axis_index("c")
    pltpu.sync_copy(g_hbm.at[pl.ds(wid * nchunk, nchunk)], idx_v)
    base = wid * nchunk * _CH
    sg = (sg0, sg1, sg2, sg3)
    so = (so0, so1, so2, so3)

    def start_gather(i, b):
        pltpu.async_copy(t_hbm.at[idx_v.at[i]], rows_v.at[b], sg[b])

    def wait_gather(b):
        pltpu.make_async_copy(t_hbm.at[pl.ds(0, _CH)], rows_v.at[b],
                              sg[b]).wait()

    def start_out(i, b):
        pltpu.async_copy(rows_v.at[b], out_hbm.at[pl.ds(base + i * _CH, _CH)],
                         so[b])

    def wait_out(b):
        pltpu.make_async_copy(rows_v.at[b], out_hbm.at[pl.ds(base, _CH)],
                              so[b]).wait()

    start_gather(0, 0)
    start_gather(1, 1)
    start_gather(2, 2)
    wait_gather(0)
    start_out(0, 0)
    start_gather(3, 3)
    wait_gather(1)
    start_out(1, 1)

    def body2(g, carry):
        for b4 in range(4):
            i = 2 + g * 4 + b4
            bcur = (2 + b4) % 4
            bnext = b4
            wait_out(bnext)
            start_gather(i + 2, bnext)
            wait_gather(bcur)
            start_out(i, bcur)
        return carry

    lax.fori_loop(0, (nchunk - 4) // 4, body2, 0)

    for i in range(2 + 4 * ((nchunk - 4) // 4), nchunk - 2):
        wait_out((i + 2) % 4)
        start_gather(i + 2, (i + 2) % 4)
        wait_gather(i % 4)
        start_out(i, i % 4)

    wait_gather((nchunk - 2) % 4)
    start_out(nchunk - 2, (nchunk - 2) % 4)
    wait_gather((nchunk - 1) % 4)
    start_out(nchunk - 1, (nchunk - 1) % 4)
    for b in range(4):
        wait_out(b)


def _sc_gather(t2, gidx2, n_tok):
    nchunk = n_tok // (_NW * _CH)
    mesh = plsc.VectorSubcoreMesh(core_axis_name="c", subcore_axis_name="s")
    fn = pl.kernel(
        functools.partial(_gather_body, nchunk),
        mesh=mesh,
        out_type=jax.ShapeDtypeStruct((n_tok, _H), jnp.float32),
        scratch_types=[
            pltpu.VMEM((nchunk, _CH), jnp.int32),
            pltpu.VMEM((4, _CH, _H), jnp.float32),
        ] + [pltpu.SemaphoreType.DMA] * 8,
    )
    return fn(t2, gidx2)


_K = 256
_TN = 3200


def _tc_body(c_ref, month_ref, week_ref, hol_ref, ww_ref,
             bw_ref, wh_ref, bh_ref, pe_ref, out_ref, stack_ref, pes_ref):
    @pl.when(pl.program_id(0) == 0)
    def _():
        wproj = jnp.dot(week_ref[...], ww_ref[...],
                        preferred_element_type=jnp.float32)
        hproj = jnp.dot(hol_ref[...], wh_ref[...],
                        preferred_element_type=jnp.float32)
        r_m = lax.broadcasted_iota(jnp.int32, (_NCOMBO, 12), 0) // 14
        c_m = lax.broadcasted_iota(jnp.int32, (_NCOMBO, 12), 1)
        r_w = (lax.broadcasted_iota(jnp.int32, (_NCOMBO, 7), 0) % 14) // 2
        c_w = lax.broadcasted_iota(jnp.int32, (_NCOMBO, 7), 1)
        r_h = lax.broadcasted_iota(jnp.int32, (_NCOMBO, 2), 0) % 2
        c_h = lax.broadcasted_iota(jnp.int32, (_NCOMBO, 2), 1)
        stack_ref[...] = (
            jnp.dot((r_m == c_m).astype(jnp.float32), month_ref[...],
                    preferred_element_type=jnp.float32)
            + jnp.dot((r_w == c_w).astype(jnp.float32), wproj,
                      preferred_element_type=jnp.float32)
            + jnp.dot((r_h == c_h).astype(jnp.float32), hproj,
                      preferred_element_type=jnp.float32))
        pebb = pe_ref[...] + bw_ref[...] + bh_ref[...]
        for j in range(_TN // _L):
            pes_ref[j * _L:(j + 1) * _L, :] = pebb

    kio = lax.broadcasted_iota(jnp.int32, (_NCOMBO, 128), 0)
    stack_bf = stack_ref[...].astype(jnp.bfloat16)
    dn = (((0,), (0,)), ((), ()))
    cv = c_ref[...]
    for r in range(_TN // 128):
        oht = (kio == cv[0, r:r + 1, :]).astype(jnp.bfloat16)
        out_ref[r * 128:(r + 1) * 128, :] = lax.dot_general(
            oht, stack_bf, dn, preferred_element_type=jnp.float32,
        ) + pes_ref[r * 128:(r + 1) * 128, :]


def _tc_lookup(c2, month, week, hol, ww, bw2, wh, bh2, pe2):
    n = c2.shape[0] * 128
    grid = n // _TN
    c3 = c2.reshape(grid, _TN // 128, 128)
    tok = pl.BlockSpec((1, _TN // 128, 128), lambda i: (i, 0, 0))
    const = lambda blk: pl.BlockSpec(blk, lambda i: tuple(0 for _ in blk))
    return pl.pallas_call(
        _tc_body,
        grid=(grid,),
        in_specs=[
            tok,
            const((12, _H)),
            const((7, _H // 2)),
            const((2, _H // 4)),
            const((_H // 2, _H)),
            const((1, _H)),
            const((_H // 4, _H)),
            const((1, _H)),
            const((_L, _H)),
        ],
        out_specs=pl.BlockSpec((_TN, _H), lambda i: (i, 0)),
        out_shape=jax.ShapeDtypeStruct((n, _H), jnp.float32),
        scratch_shapes=[pltpu.VMEM((_NCOMBO, _H), jnp.float32),
                        pltpu.VMEM((_TN, _H), jnp.float32)],
    )(c3, month, week, hol, ww, bw2, wh, bh2, pe2)


_SC_TOK = 0


def kernel(time_features, month_table, week_table, holiday_table,
           W_week, b_week, W_holiday, b_holiday, pe):
    tf = time_features.astype(jnp.int32)
    pe2 = pe[0, :_L, :]
    bw2 = b_week.reshape(1, _H)
    bh2 = b_holiday.reshape(1, _H)
    m2 = tf[..., 0].reshape(_TOK // 128, 128)
    w2 = tf[..., 1].reshape(_TOK // 128, 128)
    h2 = tf[..., 2].reshape(_TOK // 128, 128)
    pieces = []
    if _SC_TOK:
        t3 = _build_table(month_table, week_table, holiday_table,
                          W_week, bw2, W_holiday, bh2, pe2)
        t2 = t3.reshape(_L * _NCOMBO, _H)
        gidx2 = _build_idx(m2[: _SC_TOK // 128], w2[: _SC_TOK // 128],
                           h2[: _SC_TOK // 128], with_l=True)
        pieces.append(_sc_gather(t2, gidx2, _SC_TOK))
    if _SC_TOK < _TOK:
        c2 = _build_idx(m2[_SC_TOK // 128:], w2[_SC_TOK // 128:],
                        h2[_SC_TOK // 128:], with_l=False)
        pieces.append(_tc_lookup(c2, month_table, week_table,
                                 holiday_table, W_week, bw2, W_holiday, bh2,
                                 pe2))
    out2 = pieces[0] if len(pieces) == 1 else jnp.concatenate(pieces, axis=0)
    return out2.reshape(_B, _L, _H)
